# baseline (speedup 1.0000x reference)
import functools

import jax
import jax.numpy as jnp
from jax import lax
from jax.experimental import pallas as pl
from jax.experimental.pallas import tpu as pltpu
from jax.experimental.pallas import tpu_sc as plsc

N_TOKENS = 2048
D_IN = 2048
D_SAE = 16384
K = 64

INT32_MIN = -(2**31)
INT32_MAX = 2**31 - 1

NW = 32
ROWS_PER_W = N_TOKENS // NW
VECS = D_SAE // 16
GROUPS = 64
VPG = VECS // GROUPS


def _signed_key_vec(u):
    return jnp.where(u >= 0, u, jnp.bitwise_xor(jnp.bitwise_not(u), INT32_MIN))


def _sc_select_body(z_ref, thr_ref, zbuf, cand, tbuf, sbuf, sem):
    core = lax.axis_index("c")
    sub = lax.axis_index("s")
    wid = sub * 2 + core
    base = wid * ROWS_PER_W

    lane = lax.iota(jnp.int32, 16)
    ones16 = jnp.full((16,), 1, dtype=jnp.int32)

    def block_body(jb, _):
        r0 = base + jb * 4
        pltpu.sync_copy(z_ref.at[pl.ds(r0, 4)], zbuf)
        lax.fori_loop(0, 4, row_body, jb)
        return 0

    def row_body(jj, jb):
        cur = jj
        j = jb * 4 + jj

        QV = VECS // 4

        def quarter_max(q):
            def qbody(v, acc):
                return jnp.maximum(acc, zbuf[cur, pl.ds((q * QV + v) * 16, 16)])
            return lax.fori_loop(
                0, QV, qbody,
                jnp.full((16,), -jnp.inf, dtype=jnp.float32), unroll=4)

        qm = quarter_max(0)
        for q in range(1, 4):
            qm = jnp.minimum(qm, quarter_max(q))
        t0 = qm[0]
        for l in range(1, 16):
            t0 = jnp.minimum(t0, qm[l])

        def compact_body(v, ptr):
            zv = zbuf[cur, pl.ds(v * 16, 16)]
            pm = zv >= t0
            ks = _signed_key_vec(plsc.bitcast(zv, jnp.int32))
            pmi = jnp.where(pm, ones16, 0)
            c = plsc.cumsum(pmi)
            idx = ptr + c - 1
            plsc.store_scatter(cand, [idx], ks, mask=pm)
            return ptr + plsc.all_reduce_population_count(pm)[0]

        ptr = lax.fori_loop(0, VECS, compact_body, jnp.int32(0), unroll=4)

        plsc.store_scatter(cand, [ptr + lane],
                           jnp.full((16,), INT32_MIN, dtype=jnp.int32))

        nv = ptr // 16 + 1

        t0v = jnp.zeros((16,), jnp.float32) + t0
        lo0 = _signed_key_vec(plsc.bitcast(t0v, jnp.int32))[0]

        def search_body(it, carry):
            lo, hi = carry
            mid = (lo >> 1) + (hi >> 1) + (lo & hi & 1)

            def count_body(v, acc):
                kv = cand[pl.ds(v * 16, 16)]
                return acc + jnp.where(kv >= mid, ones16, 0)

            acc = lax.fori_loop(0, nv, count_body,
                                jnp.zeros((16,), jnp.int32))
            cnt = plsc.cumsum(acc)[15]
            ge = cnt >= K
            return jnp.where(ge, mid, lo), jnp.where(ge, hi, mid)

        lo, _ = lax.fori_loop(0, 32, search_body,
                              (jnp.int32(0) + lo0, jnp.int32(INT32_MAX)))

        plsc.store_scatter(tbuf, [ones16 * j], ones16 * lo, mask=lane == 0)
        return jb

    lax.fori_loop(0, ROWS_PER_W // 4, block_body, 0)
    pltpu.sync_copy(tbuf, thr_ref.at[pl.ds(base, ROWS_PER_W)])


@functools.partial(
    pl.kernel,
    out_type=jax.ShapeDtypeStruct((N_TOKENS,), jnp.int32),
    mesh=plsc.VectorSubcoreMesh(core_axis_name="c", subcore_axis_name="s"),
    compiler_params=pltpu.CompilerParams(needs_layout_passes=False),
    scratch_types=[
        pltpu.VMEM((4, D_SAE), jnp.float32),
        pltpu.VMEM((D_SAE + 16,), jnp.int32),
        pltpu.VMEM((ROWS_PER_W,), jnp.int32),
        pltpu.VMEM((16,), jnp.int32),
        pltpu.SemaphoreType.DMA,
    ],
)
def _sc_select(z_hbm, thr_hbm, zbuf, cand, tbuf, sbuf, sem):
    _sc_select_body(z_hbm, thr_hbm, zbuf, cand, tbuf, sbuf, sem)


def _encode_body(x_ref, w_ref, b_ref, z_ref):
    z = jax.lax.dot_general(
        x_ref[...], w_ref[...],
        (((1,), (1,)), ((), ())),
        preferred_element_type=jnp.float32,
    )
    z_ref[...] = z + b_ref[...]


def _decode_mask_body(z_ref, thr_ref, w_ref, b_ref, h_ref, out_ref):
    k = pl.program_id(0)

    z = z_ref[...]
    ks = _signed_key_vec(jax.lax.bitcast_convert_type(z, jnp.int32))
    h = jnp.where(ks >= thr_ref[...], jnp.maximum(z, 0.0), 0.0)
    h_ref[...] = h

    @pl.when(k == 0)
    def _():
        out_ref[...] = jnp.broadcast_to(b_ref[...], out_ref.shape)

    out_ref[...] += jax.lax.dot_general(
        h.astype(jnp.bfloat16), w_ref[...],
        (((1,), (1,)), ((), ())),
        preferred_element_type=jnp.float32,
    )


@jax.jit
def kernel(x, W_enc, b_enc, W_dec, b_dec):
    n, d_in = x.shape
    d_sae = W_enc.shape[0]

    BN = 512
    z = pl.pallas_call(
        _encode_body,
        grid=(d_sae // BN,),
        in_specs=[
            pl.BlockSpec((n, d_in), lambda j: (0, 0)),
            pl.BlockSpec((BN, d_in), lambda j: (j, 0)),
            pl.BlockSpec((1, BN), lambda j: (0, j)),
        ],
        out_specs=pl.BlockSpec((n, BN), lambda j: (0, j)),
        out_shape=jax.ShapeDtypeStruct((n, d_sae), jnp.float32),
    )(x, W_enc, b_enc.reshape(1, d_sae))

    thr = _sc_select(z)

    BK = 512
    W_dec_bf = W_dec.astype(jnp.bfloat16)
    hidden, recon = pl.pallas_call(
        _decode_mask_body,
        grid=(d_sae // BK,),
        in_specs=[
            pl.BlockSpec((n, BK), lambda k: (0, k)),
            pl.BlockSpec((n, 1), lambda k: (0, 0)),
            pl.BlockSpec((d_in, BK), lambda k: (0, k)),
            pl.BlockSpec((1, d_in), lambda k: (0, 0)),
        ],
        out_specs=[
            pl.BlockSpec((n, BK), lambda k: (0, k)),
            pl.BlockSpec((n, d_in), lambda k: (0, 0)),
        ],
        out_shape=[
            jax.ShapeDtypeStruct((n, d_sae), jnp.float32),
            jax.ShapeDtypeStruct((n, d_in), jnp.float32),
        ],
        compiler_params=pltpu.CompilerParams(
            dimension_semantics=("arbitrary",),
        ),
    )(z, thr.reshape(n, 1), W_dec_bf, b_dec.reshape(1, d_in))

    return (hidden, recon)


# speedup vs baseline: 1.1813x; 1.1813x over previous
import functools

import jax
import jax.numpy as jnp
from jax import lax
from jax.experimental import pallas as pl
from jax.experimental.pallas import tpu as pltpu
from jax.experimental.pallas import tpu_sc as plsc

N_TOKENS = 2048
D_IN = 2048
D_SAE = 16384
K = 64

INT32_MIN = -(2**31)
INT32_MAX = 2**31 - 1

NW = 32
ROWS_PER_W = N_TOKENS // NW
VECS = D_SAE // 16
GROUPS = 64
VPG = VECS // GROUPS


def _signed_key_vec(u):
    return jnp.where(u >= 0, u, jnp.bitwise_xor(jnp.bitwise_not(u), INT32_MIN))


def _sc_select_body(z_ref, thr_ref, zbuf, cand, tbuf, sbuf, sem):
    core = lax.axis_index("c")
    sub = lax.axis_index("s")
    wid = sub * 2 + core
    base = wid * ROWS_PER_W

    lane = lax.iota(jnp.int32, 16)
    ones16 = jnp.full((16,), 1, dtype=jnp.int32)

    def block_body(jb, _):
        r0 = base + jb * 4
        pltpu.sync_copy(z_ref.at[pl.ds(r0, 4)], zbuf)
        lax.fori_loop(0, 4, row_body, jb)
        return 0

    def row_body(jj, jb):
        cur = jj
        j = jb * 4 + jj

        QV = VECS // 4

        def quarter_max(q):
            def qbody(v, acc):
                return jnp.maximum(acc, zbuf[cur, pl.ds((q * QV + v) * 16, 16)])
            return lax.fori_loop(
                0, QV, qbody,
                jnp.full((16,), -jnp.inf, dtype=jnp.float32), unroll=4)

        qm = quarter_max(0)
        for q in range(1, 4):
            qm = jnp.minimum(qm, quarter_max(q))
        t0 = qm[0]
        for l in range(1, 16):
            t0 = jnp.minimum(t0, qm[l])

        def compact_body(v, ptrv):
            zv = zbuf[cur, pl.ds(v * 16, 16)]
            pm = zv >= t0
            ks = _signed_key_vec(plsc.bitcast(zv, jnp.int32))
            plsc.store_scatter(cand, [ptrv * 16 + lane], ks, mask=pm)
            return ptrv + jnp.where(pm, ones16, 0)

        ptrv = lax.fori_loop(0, VECS, compact_body,
                             jnp.zeros((16,), jnp.int32), unroll=4)

        cmax = ptrv[0]
        cmin = ptrv[0]
        for l in range(1, 16):
            cmax = jnp.maximum(cmax, ptrv[l])
            cmin = jnp.minimum(cmin, ptrv[l])

        sentinel = jnp.full((16,), INT32_MIN, dtype=jnp.int32)

        def fill_body(s, _):
            plsc.store_scatter(cand, [(ptrv + s) * 16 + lane], sentinel,
                               mask=(ptrv + s) < cmax)
            return 0

        lax.fori_loop(0, cmax - cmin, fill_body, 0)

        nv = cmax

        t0v = jnp.zeros((16,), jnp.float32) + t0
        lo0 = _signed_key_vec(plsc.bitcast(t0v, jnp.int32))[0]

        def search_body(it, carry):
            lo, hi = carry
            mid = (lo >> 1) + (hi >> 1) + (lo & hi & 1)

            def count_body(v, acc):
                kv = cand[pl.ds(v * 16, 16)]
                return acc + jnp.where(kv >= mid, ones16, 0)

            acc = lax.fori_loop(0, nv, count_body,
                                jnp.zeros((16,), jnp.int32))
            cnt = plsc.cumsum(acc)[15]
            ge = cnt >= K
            return jnp.where(ge, mid, lo), jnp.where(ge, hi, mid)

        lo, _ = lax.fori_loop(0, 32, search_body,
                              (jnp.int32(0) + lo0, jnp.int32(INT32_MAX)))

        plsc.store_scatter(tbuf, [ones16 * j], ones16 * lo, mask=lane == 0)
        return jb

    lax.fori_loop(0, ROWS_PER_W // 4, block_body, 0)
    pltpu.sync_copy(tbuf, thr_ref.at[pl.ds(base, ROWS_PER_W)])


@functools.partial(
    pl.kernel,
    out_type=jax.ShapeDtypeStruct((N_TOKENS,), jnp.int32),
    mesh=plsc.VectorSubcoreMesh(core_axis_name="c", subcore_axis_name="s"),
    compiler_params=pltpu.CompilerParams(needs_layout_passes=False),
    scratch_types=[
        pltpu.VMEM((4, D_SAE), jnp.float32),
        pltpu.VMEM((D_SAE + 16,), jnp.int32),
        pltpu.VMEM((ROWS_PER_W,), jnp.int32),
        pltpu.VMEM((16,), jnp.int32),
        pltpu.SemaphoreType.DMA,
    ],
)
def _sc_select(z_hbm, thr_hbm, zbuf, cand, tbuf, sbuf, sem):
    _sc_select_body(z_hbm, thr_hbm, zbuf, cand, tbuf, sbuf, sem)


def _encode_body(x_ref, w_ref, b_ref, z_ref):
    z = jax.lax.dot_general(
        x_ref[...], w_ref[...],
        (((1,), (1,)), ((), ())),
        preferred_element_type=jnp.float32,
    )
    z_ref[...] = z + b_ref[...]


def _decode_mask_body(z_ref, thr_ref, w_ref, b_ref, h_ref, out_ref):
    k = pl.program_id(0)

    z = z_ref[...]
    ks = _signed_key_vec(jax.lax.bitcast_convert_type(z, jnp.int32))
    h = jnp.where(ks >= thr_ref[...], jnp.maximum(z, 0.0), 0.0)
    h_ref[...] = h

    @pl.when(k == 0)
    def _():
        out_ref[...] = jnp.broadcast_to(b_ref[...], out_ref.shape)

    out_ref[...] += jax.lax.dot_general(
        h.astype(jnp.bfloat16), w_ref[...],
        (((1,), (1,)), ((), ())),
        preferred_element_type=jnp.float32,
    )


@jax.jit
def kernel(x, W_enc, b_enc, W_dec, b_dec):
    n, d_in = x.shape
    d_sae = W_enc.shape[0]

    BN = 512
    z = pl.pallas_call(
        _encode_body,
        grid=(d_sae // BN,),
        in_specs=[
            pl.BlockSpec((n, d_in), lambda j: (0, 0)),
            pl.BlockSpec((BN, d_in), lambda j: (j, 0)),
            pl.BlockSpec((1, BN), lambda j: (0, j)),
        ],
        out_specs=pl.BlockSpec((n, BN), lambda j: (0, j)),
        out_shape=jax.ShapeDtypeStruct((n, d_sae), jnp.float32),
    )(x, W_enc, b_enc.reshape(1, d_sae))

    thr = _sc_select(z)

    BK = 512
    W_dec_bf = W_dec.astype(jnp.bfloat16)
    hidden, recon = pl.pallas_call(
        _decode_mask_body,
        grid=(d_sae // BK,),
        in_specs=[
            pl.BlockSpec((n, BK), lambda k: (0, k)),
            pl.BlockSpec((n, 1), lambda k: (0, 0)),
            pl.BlockSpec((d_in, BK), lambda k: (0, k)),
            pl.BlockSpec((1, d_in), lambda k: (0, 0)),
        ],
        out_specs=[
            pl.BlockSpec((n, BK), lambda k: (0, k)),
            pl.BlockSpec((n, d_in), lambda k: (0, 0)),
        ],
        out_shape=[
            jax.ShapeDtypeStruct((n, d_sae), jnp.float32),
            jax.ShapeDtypeStruct((n, d_in), jnp.float32),
        ],
        compiler_params=pltpu.CompilerParams(
            dimension_semantics=("arbitrary",),
        ),
    )(z, thr.reshape(n, 1), W_dec_bf, b_dec.reshape(1, d_in))

    return (hidden, recon)


# speedup vs baseline: 1.3895x; 1.1763x over previous
import functools

import jax
import jax.numpy as jnp
from jax import lax
from jax.experimental import pallas as pl
from jax.experimental.pallas import tpu as pltpu
from jax.experimental.pallas import tpu_sc as plsc

N_TOKENS = 2048
D_IN = 2048
D_SAE = 16384
K = 64

INT32_MIN = -(2**31)
INT32_MAX = 2**31 - 1

NW = 32
ROWS_PER_W = N_TOKENS // NW
VECS = D_SAE // 16
GROUPS = 64
VPG = VECS // GROUPS


def _signed_key_vec(u):
    return jnp.where(u >= 0, u, jnp.bitwise_xor(jnp.bitwise_not(u), INT32_MIN))


def _sc_select_body(z_ref, thr_ref, zbuf, cand, tbuf, sbuf, sem):
    core = lax.axis_index("c")
    sub = lax.axis_index("s")
    wid = sub * 2 + core
    base = wid * ROWS_PER_W

    lane = lax.iota(jnp.int32, 16)
    ones16 = jnp.full((16,), 1, dtype=jnp.int32)

    def block_body(jb, _):
        r0 = base + jb * 4
        pltpu.sync_copy(z_ref.at[pl.ds(r0, 4)], zbuf)
        lax.fori_loop(0, 4, row_body, jb)
        return 0

    def row_body(jj, jb):
        cur = jj
        j = jb * 4 + jj

        QV = VECS // 4

        def quarter_max(q):
            def qbody(v, acc):
                return jnp.maximum(acc, zbuf[cur, pl.ds((q * QV + v) * 16, 16)])
            return lax.fori_loop(
                0, QV, qbody,
                jnp.full((16,), -jnp.inf, dtype=jnp.float32), unroll=4)

        qm = quarter_max(0)
        for q in range(1, 4):
            qm = jnp.minimum(qm, quarter_max(q))
        t0 = qm[0]
        for l in range(1, 16):
            t0 = jnp.minimum(t0, qm[l])

        def compact_body(v, ptrv):
            zv = zbuf[cur, pl.ds(v * 16, 16)]
            pm = zv >= t0
            plsc.store_scatter(cand, [ptrv * 16 + lane], zv, mask=pm)
            return ptrv + jnp.where(pm, ones16, 0)

        ptrv = lax.fori_loop(0, VECS, compact_body,
                             jnp.zeros((16,), jnp.int32), unroll=8)

        cmax = ptrv[0]
        cmin = ptrv[0]
        for l in range(1, 16):
            cmax = jnp.maximum(cmax, ptrv[l])
            cmin = jnp.minimum(cmin, ptrv[l])

        sentinel = jnp.full((16,), -jnp.inf, dtype=jnp.float32)

        def fill_body(s, _):
            plsc.store_scatter(cand, [(ptrv + s) * 16 + lane], sentinel,
                               mask=(ptrv + s) < cmax)
            return 0

        lax.fori_loop(0, cmax - cmin, fill_body, 0)

        nv = cmax

        t0v = jnp.zeros((16,), jnp.float32) + t0
        lo0 = _signed_key_vec(plsc.bitcast(t0v, jnp.int32))[0]
        SNV = 32

        def search_body(it, carry):
            lo, hi = carry
            mid = (lo >> 1) + (hi >> 1) + (lo & hi & 1)
            midv = jnp.zeros((16,), jnp.int32) + mid
            umid = jnp.where(midv >= 0, midv,
                             jnp.bitwise_not(jnp.bitwise_xor(midv, INT32_MIN)))
            fmid = plsc.bitcast(umid, jnp.float32)

            acc = jnp.zeros((16,), jnp.int32)
            for v in range(SNV):
                zvld = cand[pl.ds(v * 16, 16)]
                inb = (zvld >= fmid) & (v < nv)
                acc = acc + jnp.where(inb, ones16, 0)

            def tail_body(v, a):
                zvld = cand[pl.ds(v * 16, 16)]
                return a + jnp.where(zvld >= fmid, ones16, 0)

            acc = lax.fori_loop(SNV, jnp.maximum(nv, SNV), tail_body, acc)
            cnt = plsc.cumsum(acc)[15]
            ge = cnt >= K
            return jnp.where(ge, mid, lo), jnp.where(ge, hi, mid)

        lo, _ = lax.fori_loop(0, 32, search_body,
                              (jnp.int32(0) + lo0, jnp.int32(INT32_MAX)))

        plsc.store_scatter(tbuf, [ones16 * j], ones16 * lo, mask=lane == 0)
        return jb

    lax.fori_loop(0, ROWS_PER_W // 4, block_body, 0)
    pltpu.sync_copy(tbuf, thr_ref.at[pl.ds(base, ROWS_PER_W)])


@functools.partial(
    pl.kernel,
    out_type=jax.ShapeDtypeStruct((N_TOKENS,), jnp.int32),
    mesh=plsc.VectorSubcoreMesh(core_axis_name="c", subcore_axis_name="s"),
    compiler_params=pltpu.CompilerParams(needs_layout_passes=False),
    scratch_types=[
        pltpu.VMEM((4, D_SAE), jnp.float32),
        pltpu.VMEM((D_SAE + 16,), jnp.float32),
        pltpu.VMEM((ROWS_PER_W,), jnp.int32),
        pltpu.VMEM((16,), jnp.int32),
        pltpu.SemaphoreType.DMA,
    ],
)
def _sc_select(z_hbm, thr_hbm, zbuf, cand, tbuf, sbuf, sem):
    _sc_select_body(z_hbm, thr_hbm, zbuf, cand, tbuf, sbuf, sem)


def _encode_body(x_ref, w_ref, b_ref, z_ref):
    z = jax.lax.dot_general(
        x_ref[...], w_ref[...],
        (((1,), (1,)), ((), ())),
        preferred_element_type=jnp.float32,
    )
    z_ref[...] = z + b_ref[...]


def _decode_mask_body(z_ref, thr_ref, w_ref, b_ref, h_ref, out_ref):
    k = pl.program_id(0)

    z = z_ref[...]
    ks = _signed_key_vec(jax.lax.bitcast_convert_type(z, jnp.int32))
    h = jnp.where(ks >= thr_ref[...], jnp.maximum(z, 0.0), 0.0)
    h_ref[...] = h

    @pl.when(k == 0)
    def _():
        out_ref[...] = jnp.broadcast_to(b_ref[...], out_ref.shape)

    out_ref[...] += jax.lax.dot_general(
        h.astype(jnp.bfloat16), w_ref[...],
        (((1,), (1,)), ((), ())),
        preferred_element_type=jnp.float32,
    )


@jax.jit
def kernel(x, W_enc, b_enc, W_dec, b_dec):
    n, d_in = x.shape
    d_sae = W_enc.shape[0]

    BN = 512
    z = pl.pallas_call(
        _encode_body,
        grid=(d_sae // BN,),
        in_specs=[
            pl.BlockSpec((n, d_in), lambda j: (0, 0)),
            pl.BlockSpec((BN, d_in), lambda j: (j, 0)),
            pl.BlockSpec((1, BN), lambda j: (0, j)),
        ],
        out_specs=pl.BlockSpec((n, BN), lambda j: (0, j)),
        out_shape=jax.ShapeDtypeStruct((n, d_sae), jnp.float32),
    )(x, W_enc, b_enc.reshape(1, d_sae))

    thr = _sc_select(z)

    BK = 512
    W_dec_bf = W_dec.astype(jnp.bfloat16)
    hidden, recon = pl.pallas_call(
        _decode_mask_body,
        grid=(d_sae // BK,),
        in_specs=[
            pl.BlockSpec((n, BK), lambda k: (0, k)),
            pl.BlockSpec((n, 1), lambda k: (0, 0)),
            pl.BlockSpec((d_in, BK), lambda k: (0, k)),
            pl.BlockSpec((1, d_in), lambda k: (0, 0)),
        ],
        out_specs=[
            pl.BlockSpec((n, BK), lambda k: (0, k)),
            pl.BlockSpec((n, d_in), lambda k: (0, 0)),
        ],
        out_shape=[
            jax.ShapeDtypeStruct((n, d_sae), jnp.float32),
            jax.ShapeDtypeStruct((n, d_in), jnp.float32),
        ],
        compiler_params=pltpu.CompilerParams(
            dimension_semantics=("arbitrary",),
        ),
    )(z, thr.reshape(n, 1), W_dec_bf, b_dec.reshape(1, d_in))

    return (hidden, recon)


# speedup vs baseline: 2.0711x; 1.4905x over previous
import functools

import jax
import jax.numpy as jnp
from jax import lax
from jax.experimental import pallas as pl
from jax.experimental.pallas import tpu as pltpu
from jax.experimental.pallas import tpu_sc as plsc

N_TOKENS = 2048
D_IN = 2048
D_SAE = 16384
K = 64

INT32_MIN = -(2**31)
INT32_MAX = 2**31 - 1

NW = 32
SC_ROWS = 1024
ROWS_PER_W = SC_ROWS // NW
VECS = D_SAE // 16
GROUPS = 64
VPG = VECS // GROUPS


def _signed_key_vec(u):
    return jnp.where(u >= 0, u, jnp.bitwise_xor(jnp.bitwise_not(u), INT32_MIN))


def _sc_select_body(z_ref, thr_ref, zbuf, cand, tbuf, sbuf, sem):
    core = lax.axis_index("c")
    sub = lax.axis_index("s")
    wid = sub * 2 + core
    base = wid * ROWS_PER_W

    lane = lax.iota(jnp.int32, 16)
    ones16 = jnp.full((16,), 1, dtype=jnp.int32)

    def block_body(jb, _):
        r0 = base + jb * 4
        pltpu.sync_copy(z_ref.at[pl.ds(r0, 4)], zbuf)
        lax.fori_loop(0, 4, row_body, jb)
        return 0

    def row_body(jj, jb):
        cur = jj
        j = jb * 4 + jj

        QV = VECS // 4

        def quarter_max(q):
            def qbody(v, acc):
                return jnp.maximum(acc, zbuf[cur, pl.ds((q * QV + v) * 16, 16)])
            return lax.fori_loop(
                0, QV, qbody,
                jnp.full((16,), -jnp.inf, dtype=jnp.float32), unroll=4)

        qm = quarter_max(0)
        for q in range(1, 4):
            qm = jnp.minimum(qm, quarter_max(q))
        t0 = qm[0]
        for l in range(1, 16):
            t0 = jnp.minimum(t0, qm[l])

        def compact_body(v, ptrv):
            zv = zbuf[cur, pl.ds(v * 16, 16)]
            pm = zv >= t0
            plsc.store_scatter(cand, [ptrv * 16 + lane], zv, mask=pm)
            return ptrv + jnp.where(pm, ones16, 0)

        ptrv = lax.fori_loop(0, VECS, compact_body,
                             jnp.zeros((16,), jnp.int32), unroll=8)

        cmax = ptrv[0]
        cmin = ptrv[0]
        for l in range(1, 16):
            cmax = jnp.maximum(cmax, ptrv[l])
            cmin = jnp.minimum(cmin, ptrv[l])

        sentinel = jnp.full((16,), -jnp.inf, dtype=jnp.float32)

        def fill_body(s, _):
            plsc.store_scatter(cand, [(ptrv + s) * 16 + lane], sentinel,
                               mask=(ptrv + s) < cmax)
            return 0

        lax.fori_loop(0, cmax - cmin, fill_body, 0)

        nv = cmax

        t0v = jnp.zeros((16,), jnp.float32) + t0
        lo0 = _signed_key_vec(plsc.bitcast(t0v, jnp.int32))[0]
        SNV = 32

        def search_body(it, carry):
            lo, hi = carry
            mid = (lo >> 1) + (hi >> 1) + (lo & hi & 1)
            midv = jnp.zeros((16,), jnp.int32) + mid
            umid = jnp.where(midv >= 0, midv,
                             jnp.bitwise_not(jnp.bitwise_xor(midv, INT32_MIN)))
            fmid = plsc.bitcast(umid, jnp.float32)

            acc = jnp.zeros((16,), jnp.int32)
            for v in range(SNV):
                zvld = cand[pl.ds(v * 16, 16)]
                inb = (zvld >= fmid) & (v < nv)
                acc = acc + jnp.where(inb, ones16, 0)

            def tail_body(v, a):
                zvld = cand[pl.ds(v * 16, 16)]
                return a + jnp.where(zvld >= fmid, ones16, 0)

            acc = lax.fori_loop(SNV, jnp.maximum(nv, SNV), tail_body, acc)
            cnt = plsc.cumsum(acc)[15]
            ge = cnt >= K
            return jnp.where(ge, mid, lo), jnp.where(ge, hi, mid)

        lo, _ = lax.fori_loop(0, 32, search_body,
                              (jnp.int32(0) + lo0, jnp.int32(INT32_MAX)))

        plsc.store_scatter(tbuf, [ones16 * j], ones16 * lo, mask=lane == 0)
        return jb

    lax.fori_loop(0, ROWS_PER_W // 4, block_body, 0)
    pltpu.sync_copy(tbuf, thr_ref.at[pl.ds(base, ROWS_PER_W)])


@functools.partial(
    pl.kernel,
    out_type=jax.ShapeDtypeStruct((SC_ROWS,), jnp.int32),
    mesh=plsc.VectorSubcoreMesh(core_axis_name="c", subcore_axis_name="s"),
    compiler_params=pltpu.CompilerParams(needs_layout_passes=False),
    scratch_types=[
        pltpu.VMEM((4, D_SAE), jnp.float32),
        pltpu.VMEM((D_SAE + 16,), jnp.float32),
        pltpu.VMEM((ROWS_PER_W,), jnp.int32),
        pltpu.VMEM((16,), jnp.int32),
        pltpu.SemaphoreType.DMA,
    ],
)
def _sc_select(z_hbm, thr_hbm, zbuf, cand, tbuf, sbuf, sem):
    _sc_select_body(z_hbm, thr_hbm, zbuf, cand, tbuf, sbuf, sem)


def _tc_select_body(z_ref, t_ref):
    z = z_ref[...]
    key = _signed_key_vec(jax.lax.bitcast_convert_type(z, jnp.int32))
    br = z.shape[0]

    def body(i, carry):
        lo, hi = carry
        mid = (lo >> 1) + (hi >> 1) + (lo & hi & 1)
        cnt = jnp.sum((key >= mid).astype(jnp.int32), axis=1, keepdims=True)
        ge = cnt >= K
        return jnp.where(ge, mid, lo), jnp.where(ge, hi, mid)

    lo0 = jnp.full((br, 1), INT32_MIN, dtype=jnp.int32)
    hi0 = jnp.full((br, 1), INT32_MAX, dtype=jnp.int32)
    t, _ = jax.lax.fori_loop(0, 32, body, (lo0, hi0))
    t_ref[...] = t


def _encode_body(x_ref, w_ref, b_ref, z_ref):
    z = jax.lax.dot_general(
        x_ref[...], w_ref[...],
        (((1,), (1,)), ((), ())),
        preferred_element_type=jnp.float32,
    )
    z_ref[...] = z + b_ref[...]


def _decode_mask_body(z_ref, thr_ref, w_ref, b_ref, h_ref, out_ref):
    k = pl.program_id(0)

    z = z_ref[...]
    ks = _signed_key_vec(jax.lax.bitcast_convert_type(z, jnp.int32))
    h = jnp.where(ks >= thr_ref[...], jnp.maximum(z, 0.0), 0.0)
    h_ref[...] = h

    @pl.when(k == 0)
    def _():
        out_ref[...] = jnp.broadcast_to(b_ref[...], out_ref.shape)

    out_ref[...] += jax.lax.dot_general(
        h.astype(jnp.bfloat16), w_ref[...],
        (((1,), (1,)), ((), ())),
        preferred_element_type=jnp.float32,
    )


@jax.jit
def kernel(x, W_enc, b_enc, W_dec, b_dec):
    n, d_in = x.shape
    d_sae = W_enc.shape[0]

    BN = 512
    z = pl.pallas_call(
        _encode_body,
        grid=(d_sae // BN,),
        in_specs=[
            pl.BlockSpec((n, d_in), lambda j: (0, 0)),
            pl.BlockSpec((BN, d_in), lambda j: (j, 0)),
            pl.BlockSpec((1, BN), lambda j: (0, j)),
        ],
        out_specs=pl.BlockSpec((n, BN), lambda j: (0, j)),
        out_shape=jax.ShapeDtypeStruct((n, d_sae), jnp.float32),
    )(x, W_enc, b_enc.reshape(1, d_sae))

    thr_sc = _sc_select(z)

    BR = 128
    n_tc = n - SC_ROWS
    thr_tc = pl.pallas_call(
        _tc_select_body,
        grid=(n_tc // BR,),
        in_specs=[pl.BlockSpec((BR, d_sae), lambda i, o=SC_ROWS // BR: (i + o, 0))],
        out_specs=pl.BlockSpec((BR, 1), lambda i: (i, 0)),
        out_shape=jax.ShapeDtypeStruct((n_tc, 1), jnp.int32),
    )(z)

    thr = jnp.concatenate([thr_sc.reshape(SC_ROWS, 1), thr_tc], axis=0)

    BK = 512
    W_dec_bf = W_dec.astype(jnp.bfloat16)
    hidden, recon = pl.pallas_call(
        _decode_mask_body,
        grid=(d_sae // BK,),
        in_specs=[
            pl.BlockSpec((n, BK), lambda k: (0, k)),
            pl.BlockSpec((n, 1), lambda k: (0, 0)),
            pl.BlockSpec((d_in, BK), lambda k: (0, k)),
            pl.BlockSpec((1, d_in), lambda k: (0, 0)),
        ],
        out_specs=[
            pl.BlockSpec((n, BK), lambda k: (0, k)),
            pl.BlockSpec((n, d_in), lambda k: (0, 0)),
        ],
        out_shape=[
            jax.ShapeDtypeStruct((n, d_sae), jnp.float32),
            jax.ShapeDtypeStruct((n, d_in), jnp.float32),
        ],
        compiler_params=pltpu.CompilerParams(
            dimension_semantics=("arbitrary",),
        ),
    )(z, thr, W_dec_bf, b_dec.reshape(1, d_in))

    return (hidden, recon)


# speedup vs baseline: 2.0720x; 1.0004x over previous
import functools

import jax
import jax.numpy as jnp
from jax import lax
from jax.experimental import pallas as pl
from jax.experimental.pallas import tpu as pltpu
from jax.experimental.pallas import tpu_sc as plsc

N_TOKENS = 2048
D_IN = 2048
D_SAE = 16384
K = 64

INT32_MIN = -(2**31)
INT32_MAX = 2**31 - 1

NW = 32
SC_ROWS = 1024
ROWS_PER_W = SC_ROWS // NW
VECS = D_SAE // 16


def _signed_key_vec(u):
    return jnp.where(u >= 0, u, jnp.bitwise_xor(jnp.bitwise_not(u), INT32_MIN))


def _sc_select_body(z_ref, thr_ref, zbuf, cand, tbuf):
    core = lax.axis_index("c")
    sub = lax.axis_index("s")
    wid = sub * 2 + core
    base = wid * ROWS_PER_W

    lane = lax.iota(jnp.int32, 16)
    ones16 = jnp.full((16,), 1, dtype=jnp.int32)

    def block_body(jb, _):
        r0 = base + jb * 4
        pltpu.sync_copy(z_ref.at[pl.ds(r0, 4)], zbuf)
        lax.fori_loop(0, 4, row_body, jb)
        return 0

    def row_body(jj, jb):
        cur = jj
        j = jb * 4 + jj

        QV = VECS // 4

        def quarter_max(q):
            def qbody(v, acc):
                return jnp.maximum(acc, zbuf[cur, pl.ds((q * QV + v) * 16, 16)])
            return lax.fori_loop(
                0, QV, qbody,
                jnp.full((16,), -jnp.inf, dtype=jnp.float32), unroll=4)

        qm = quarter_max(0)
        for q in range(1, 4):
            qm = jnp.minimum(qm, quarter_max(q))
        t0 = qm[0]
        for l in range(1, 16):
            t0 = jnp.minimum(t0, qm[l])

        def compact_body(v, ptrv):
            zv = zbuf[cur, pl.ds(v * 16, 16)]
            pm = zv >= t0
            plsc.store_scatter(cand, [ptrv * 16 + lane], zv, mask=pm)
            return ptrv + jnp.where(pm, ones16, 0)

        ptrv = lax.fori_loop(0, VECS, compact_body,
                             jnp.zeros((16,), jnp.int32), unroll=8)

        cmax = ptrv[0]
        cmin = ptrv[0]
        for l in range(1, 16):
            cmax = jnp.maximum(cmax, ptrv[l])
            cmin = jnp.minimum(cmin, ptrv[l])

        sentinel = jnp.full((16,), -jnp.inf, dtype=jnp.float32)

        def fill_body(s, _):
            plsc.store_scatter(cand, [(ptrv + s) * 16 + lane], sentinel,
                               mask=(ptrv + s) < cmax)
            return 0

        lax.fori_loop(0, cmax - cmin, fill_body, 0)

        nv = cmax

        t0v = jnp.zeros((16,), jnp.float32) + t0
        lo0 = _signed_key_vec(plsc.bitcast(t0v, jnp.int32))[0]
        SNV = 32

        def search_body(it, carry):
            lo, hi = carry
            mid = (lo >> 1) + (hi >> 1) + (lo & hi & 1)
            midv = jnp.zeros((16,), jnp.int32) + mid
            umid = jnp.where(midv >= 0, midv,
                             jnp.bitwise_not(jnp.bitwise_xor(midv, INT32_MIN)))
            fmid = plsc.bitcast(umid, jnp.float32)

            acc = jnp.zeros((16,), jnp.int32)
            for v in range(SNV):
                zvld = cand[pl.ds(v * 16, 16)]
                inb = (zvld >= fmid) & (v < nv)
                acc = acc + jnp.where(inb, ones16, 0)

            def tail_body(v, a):
                zvld = cand[pl.ds(v * 16, 16)]
                return a + jnp.where(zvld >= fmid, ones16, 0)

            acc = lax.fori_loop(SNV, jnp.maximum(nv, SNV), tail_body, acc)
            cnt = plsc.cumsum(acc)[15]
            ge = cnt >= K
            return jnp.where(ge, mid, lo), jnp.where(ge, hi, mid)

        lo, _ = lax.fori_loop(0, 32, search_body,
                              (jnp.int32(0) + lo0, jnp.int32(INT32_MAX)))

        plsc.store_scatter(tbuf, [ones16 * j], ones16 * lo, mask=lane == 0)
        return jb

    lax.fori_loop(0, ROWS_PER_W // 4, block_body, 0)
    pltpu.sync_copy(tbuf, thr_ref.at[pl.ds(base, ROWS_PER_W)])


@functools.partial(
    pl.kernel,
    out_type=jax.ShapeDtypeStruct((SC_ROWS,), jnp.int32),
    mesh=plsc.VectorSubcoreMesh(core_axis_name="c", subcore_axis_name="s"),
    compiler_params=pltpu.CompilerParams(needs_layout_passes=False),
    scratch_types=[
        pltpu.VMEM((4, D_SAE), jnp.float32),
        pltpu.VMEM((D_SAE + 16,), jnp.float32),
        pltpu.VMEM((ROWS_PER_W,), jnp.int32),
    ],
)
def _sc_select(z_hbm, thr_hbm, zbuf, cand, tbuf):
    _sc_select_body(z_hbm, thr_hbm, zbuf, cand, tbuf)


def _tc_select_body(z_ref, t_ref):
    z = z_ref[...]
    key = _signed_key_vec(jax.lax.bitcast_convert_type(z, jnp.int32))
    br = z.shape[0]

    def body(i, carry):
        lo, hi = carry
        mid = (lo >> 1) + (hi >> 1) + (lo & hi & 1)
        cnt = jnp.sum((key >= mid).astype(jnp.int32), axis=1, keepdims=True)
        ge = cnt >= K
        return jnp.where(ge, mid, lo), jnp.where(ge, hi, mid)

    lo0 = jnp.full((br, 1), INT32_MIN, dtype=jnp.int32)
    hi0 = jnp.full((br, 1), INT32_MAX, dtype=jnp.int32)
    t, _ = jax.lax.fori_loop(0, 32, body, (lo0, hi0))
    t_ref[...] = t


def _encode_body(x_ref, w_ref, b_ref, z_ref):
    z = jax.lax.dot_general(
        x_ref[...], w_ref[...],
        (((1,), (1,)), ((), ())),
        preferred_element_type=jnp.float32,
    )
    z_ref[...] = z + b_ref[...]


def _decode_mask_body(z_ref, thr_ref, w_ref, b_ref, h_ref, out_ref):
    k = pl.program_id(0)

    z = z_ref[...]
    ks = _signed_key_vec(jax.lax.bitcast_convert_type(z, jnp.int32))
    h = jnp.where(ks >= thr_ref[...], jnp.maximum(z, 0.0), 0.0)
    h_ref[...] = h

    @pl.when(k == 0)
    def _():
        out_ref[...] = jnp.broadcast_to(b_ref[...], out_ref.shape)

    out_ref[...] += jax.lax.dot_general(
        h.astype(jnp.bfloat16), w_ref[...],
        (((1,), (1,)), ((), ())),
        preferred_element_type=jnp.float32,
    )


@jax.jit
def kernel(x, W_enc, b_enc, W_dec, b_dec):
    n, d_in = x.shape
    d_sae = W_enc.shape[0]

    BN = 512
    z = pl.pallas_call(
        _encode_body,
        grid=(d_sae // BN,),
        in_specs=[
            pl.BlockSpec((n, d_in), lambda j: (0, 0)),
            pl.BlockSpec((BN, d_in), lambda j: (j, 0)),
            pl.BlockSpec((1, BN), lambda j: (0, j)),
        ],
        out_specs=pl.BlockSpec((n, BN), lambda j: (0, j)),
        out_shape=jax.ShapeDtypeStruct((n, d_sae), jnp.float32),
    )(x, W_enc, b_enc.reshape(1, d_sae))

    thr_sc = _sc_select(z)

    BR = 128
    n_tc = n - SC_ROWS
    thr_tc = pl.pallas_call(
        _tc_select_body,
        grid=(n_tc // BR,),
        in_specs=[pl.BlockSpec((BR, d_sae), lambda i, o=SC_ROWS // BR: (i + o, 0))],
        out_specs=pl.BlockSpec((BR, 1), lambda i: (i, 0)),
        out_shape=jax.ShapeDtypeStruct((n_tc, 1), jnp.int32),
    )(z)

    thr = jnp.concatenate([thr_sc.reshape(SC_ROWS, 1), thr_tc], axis=0)

    BK = 512
    W_dec_bf = W_dec.astype(jnp.bfloat16)
    hidden, recon = pl.pallas_call(
        _decode_mask_body,
        grid=(d_sae // BK,),
        in_specs=[
            pl.BlockSpec((n, BK), lambda k: (0, k)),
            pl.BlockSpec((n, 1), lambda k: (0, 0)),
            pl.BlockSpec((d_in, BK), lambda k: (0, k)),
            pl.BlockSpec((1, d_in), lambda k: (0, 0)),
        ],
        out_specs=[
            pl.BlockSpec((n, BK), lambda k: (0, k)),
            pl.BlockSpec((n, d_in), lambda k: (0, 0)),
        ],
        out_shape=[
            jax.ShapeDtypeStruct((n, d_sae), jnp.float32),
            jax.ShapeDtypeStruct((n, d_in), jnp.float32),
        ],
        compiler_params=pltpu.CompilerParams(
            dimension_semantics=("arbitrary",),
        ),
    )(z, thr, W_dec_bf, b_dec.reshape(1, d_in))

    return (hidden, recon)
